# baseline (speedup 1.0000x reference)
import functools

import jax
import jax.numpy as jnp
from jax import lax
from jax.experimental import pallas as pl
from jax.experimental.pallas import tpu as pltpu
from jax.experimental.pallas import tpu_sc as plsc

_VOCAB = 1000
_K = 8
_EMBED = 32
_B = 4096
_HID = 512
_OUT = 1000

_ROW_W = _K * _VOCAB
_BB = 512

_NC = 2
_NS = 16
_NW = _NC * _NS

_DS = 1024
_C0 = _B - _DS
_CG = 128
_NTASK = (_DS // _CG) * _K
_TPW = _NTASK // _NW
_H0 = 504
_H1 = _VOCAB - _H0
_NLG = _CG // 16
_RUNROLL = 4


def _rows_scan(buf, nrows, rbase, carry):
    def body(i, carry):
        ms = list(carry[:_NLG])
        ps = list(carry[_NLG:2 * _NLG])
        rvec = carry[-1]
        for u in range(_RUNROLL):
            r = i * _RUNROLL + u
            for lg in range(_NLG):
                v = buf[r, pl.ds(lg * 16, 16)]
                upd = v > ms[lg]
                ms[lg] = jnp.where(upd, v, ms[lg])
                ps[lg] = jnp.where(upd, rvec, ps[lg])
            rvec = rvec + 1
        return tuple(ms) + tuple(ps) + (rvec,)

    return lax.fori_loop(0, nrows // _RUNROLL, body, carry)


@functools.lru_cache(maxsize=1)
def _make_sc_argmax():
    mesh = plsc.VectorSubcoreMesh(core_axis_name="c", subcore_axis_name="s")
    assert _H0 % _RUNROLL == 0 and _H1 % _RUNROLL == 0

    @functools.partial(
        pl.kernel,
        mesh=mesh,
        out_type=jax.ShapeDtypeStruct((_K, _DS), jnp.int32),
        scratch_types=[
            pltpu.VMEM((_H0, _CG), jnp.float32),
            pltpu.VMEM((_H1, _CG), jnp.float32),
            pltpu.VMEM((_CG,), jnp.int32),
            pltpu.SemaphoreType.DMA,
            pltpu.SemaphoreType.DMA,
        ],
        compiler_params=pltpu.CompilerParams(
            use_tc_tiling_on_sc=True, needs_layout_passes=False),
    )
    def sc_argmax(xt_hbm, idx_hbm, bufa, bufb, idx_v, sema, semb):
        wid = lax.axis_index("s") * _NC + lax.axis_index("c")

        def src(t, half):
            g = wid * _TPW + t
            cg = g // _K
            k = g - cg * _K
            r0 = k * _VOCAB + (0 if half == 0 else _H0)
            nr = _H0 if half == 0 else _H1
            return k, cg, xt_hbm.at[pl.ds(r0, nr),
                                    pl.ds(_C0 + cg * _CG, _CG)]

        neg = jnp.full((16,), -jnp.inf, dtype=jnp.float32)
        zero = jnp.zeros((16,), dtype=jnp.int32)
        cpa = pltpu.async_copy(src(0, 0)[2], bufa, sema)
        cpb = pltpu.async_copy(src(0, 1)[2], bufb, semb)
        for t in range(_TPW):
            carry = (neg,) * _NLG + (zero,) * _NLG + (zero,)
            cpa.wait()
            carry = _rows_scan(bufa, _H0, 0, carry)
            if t + 1 < _TPW:
                cpa = pltpu.async_copy(src(t + 1, 0)[2], bufa, sema)
            cpb.wait()
            carry = _rows_scan(bufb, _H1, _H0, carry)
            if t + 1 < _TPW:
                cpb = pltpu.async_copy(src(t + 1, 1)[2], bufb, semb)
            for lg in range(_NLG):
                idx_v[pl.ds(lg * 16, 16)] = carry[_NLG + lg]
            kt, cgt, _ = src(t, 0)
            pltpu.sync_copy(idx_v, idx_hbm.at[kt, pl.ds(cgt * _CG, _CG)])

    return sc_argmax


def _mlp_t(fe_t, w1_ref, b1_ref, w2t_ref, b2_ref, o_ref):
    h = lax.dot_general(w1_ref[...], fe_t, (((0,), (0,)), ((), ())),
                        preferred_element_type=jnp.float32)
    h = jnp.maximum(h + b1_ref[...], 0.0)
    o = jnp.dot(w2t_ref[...], h, preferred_element_type=jnp.float32)
    o_ref[...] = o + b2_ref[...]


def _fused_body(xt_ref, embt_ref, w1_ref, b1_ref, w2t_ref, b2_ref, o_ref):
    xb = xt_ref[...]
    embt = embt_ref[...]
    fe_parts = []
    for k in range(_K):
        xk = xb[k * _VOCAB:(k + 1) * _VOCAB, :]
        m = jnp.max(xk, axis=0, keepdims=True)
        ii = lax.broadcasted_iota(jnp.int32, xk.shape, 0)
        cand = jnp.where(xk == m, ii, _VOCAB)
        idxk = jnp.min(cand, axis=0, keepdims=True)
        onehot = (ii == idxk).astype(jnp.float32)
        fe_parts.append(
            jnp.dot(embt, onehot, preferred_element_type=jnp.float32))
    fe_t = jnp.concatenate(fe_parts, axis=0)
    _mlp_t(fe_t, w1_ref, b1_ref, w2t_ref, b2_ref, o_ref)


def _fused(xt, embt, w1, b1c, w2t, b2c, ncols):
    grid = ncols // _BB
    return pl.pallas_call(
        _fused_body,
        grid=(grid,),
        in_specs=[
            pl.BlockSpec((_ROW_W, _BB), lambda i: (0, i)),
            pl.BlockSpec((_EMBED, _VOCAB), lambda i: (0, 0)),
            pl.BlockSpec((_K * _EMBED, _HID), lambda i: (0, 0)),
            pl.BlockSpec((_HID, 1), lambda i: (0, 0)),
            pl.BlockSpec((_OUT, _HID), lambda i: (0, 0)),
            pl.BlockSpec((_OUT, 1), lambda i: (0, 0)),
        ],
        out_specs=pl.BlockSpec((_OUT, _BB), lambda i: (0, i)),
        out_shape=jax.ShapeDtypeStruct((_OUT, _B), jnp.float32),
    )(xt, embt, w1, b1c, w2t, b2c)


def _tail_body(prev_ref, idx_ref, embt_ref, w1_ref, b1_ref, w2t_ref, b2_ref,
               o_ref):
    del prev_ref
    idxb = idx_ref[...]
    embt = embt_ref[...]
    fe_parts = []
    for k in range(_K):
        idxk = idxb[k:k + 1, :]
        ii = lax.broadcasted_iota(jnp.int32, (_VOCAB, idxb.shape[1]), 0)
        onehot = (ii == idxk).astype(jnp.float32)
        fe_parts.append(
            jnp.dot(embt, onehot, preferred_element_type=jnp.float32))
    fe_t = jnp.concatenate(fe_parts, axis=0)
    _mlp_t(fe_t, w1_ref, b1_ref, w2t_ref, b2_ref, o_ref)


def _tail_mlp(prev_out_t, idx_t, embt, w1, b1c, w2t, b2c):
    grid = _DS // _BB
    off = _C0 // _BB
    return pl.pallas_call(
        _tail_body,
        grid=(grid,),
        in_specs=[
            pl.BlockSpec(memory_space=pl.ANY),
            pl.BlockSpec((_K, _BB), lambda i: (0, i)),
            pl.BlockSpec((_EMBED, _VOCAB), lambda i: (0, 0)),
            pl.BlockSpec((_K * _EMBED, _HID), lambda i: (0, 0)),
            pl.BlockSpec((_HID, 1), lambda i: (0, 0)),
            pl.BlockSpec((_OUT, _HID), lambda i: (0, 0)),
            pl.BlockSpec((_OUT, 1), lambda i: (0, 0)),
        ],
        out_specs=pl.BlockSpec((_OUT, _BB), lambda i: (0, off + i)),
        out_shape=jax.ShapeDtypeStruct((_OUT, _B), jnp.float32),
        input_output_aliases={0: 0},
    )(prev_out_t, idx_t, embt, w1, b1c, w2t, b2c)


def kernel(x, emb, W1, b1, W2, b2):
    xt = x.T
    embt = emb.T
    w2t = W2.T
    b1c = b1.reshape(_HID, 1)
    b2c = b2.reshape(_OUT, 1)
    idx_t = _make_sc_argmax()(xt)
    head_t = _fused(xt, embt, W1, b1c, w2t, b2c, _C0)
    out_t = _tail_mlp(head_t, idx_t, embt, W1, b1c, w2t, b2c)
    return out_t.T


# speedup vs baseline: 1.0169x; 1.0169x over previous
import functools

import jax
import jax.numpy as jnp
from jax import lax
from jax.experimental import pallas as pl
from jax.experimental.pallas import tpu as pltpu
from jax.experimental.pallas import tpu_sc as plsc

_VOCAB = 1000
_K = 8
_EMBED = 32
_B = 4096
_HID = 512
_OUT = 1000

_ROW_W = _K * _VOCAB
_BB = 512

_NC = 2
_NS = 16
_NW = _NC * _NS

_DS = 512
_C0 = _B - _DS
_CG = 128
_NTASK = (_DS // _CG) * _K
_TPW = _NTASK // _NW
_H0 = 504
_H1 = _VOCAB - _H0
_NLG = _CG // 16
_RUNROLL = 4


def _rows_scan(buf, nrows, rbase, carry):
    def body(i, carry):
        ms = list(carry[:_NLG])
        ps = list(carry[_NLG:2 * _NLG])
        rvec = carry[-1]
        for u in range(_RUNROLL):
            r = i * _RUNROLL + u
            for lg in range(_NLG):
                v = buf[r, pl.ds(lg * 16, 16)]
                upd = v > ms[lg]
                ms[lg] = jnp.where(upd, v, ms[lg])
                ps[lg] = jnp.where(upd, rvec, ps[lg])
            rvec = rvec + 1
        return tuple(ms) + tuple(ps) + (rvec,)

    return lax.fori_loop(0, nrows // _RUNROLL, body, carry)


@functools.lru_cache(maxsize=1)
def _make_sc_argmax():
    mesh = plsc.VectorSubcoreMesh(core_axis_name="c", subcore_axis_name="s")
    assert _H0 % _RUNROLL == 0 and _H1 % _RUNROLL == 0

    @functools.partial(
        pl.kernel,
        mesh=mesh,
        out_type=jax.ShapeDtypeStruct((_K, _DS), jnp.int32),
        scratch_types=[
            pltpu.VMEM((_H0, _CG), jnp.float32),
            pltpu.VMEM((_H1, _CG), jnp.float32),
            pltpu.VMEM((_CG,), jnp.int32),
            pltpu.SemaphoreType.DMA,
            pltpu.SemaphoreType.DMA,
        ],
        compiler_params=pltpu.CompilerParams(
            use_tc_tiling_on_sc=True, needs_layout_passes=False),
    )
    def sc_argmax(xt_hbm, idx_hbm, bufa, bufb, idx_v, sema, semb):
        wid = lax.axis_index("s") * _NC + lax.axis_index("c")

        def src(t, half):
            g = wid * _TPW + t
            cg = g // _K
            k = g - cg * _K
            r0 = k * _VOCAB + (0 if half == 0 else _H0)
            nr = _H0 if half == 0 else _H1
            return k, cg, xt_hbm.at[pl.ds(r0, nr),
                                    pl.ds(_C0 + cg * _CG, _CG)]

        neg = jnp.full((16,), -jnp.inf, dtype=jnp.float32)
        zero = jnp.zeros((16,), dtype=jnp.int32)
        cpa = pltpu.async_copy(src(0, 0)[2], bufa, sema)
        cpb = pltpu.async_copy(src(0, 1)[2], bufb, semb)
        for t in range(_TPW):
            carry = (neg,) * _NLG + (zero,) * _NLG + (zero,)
            cpa.wait()
            carry = _rows_scan(bufa, _H0, 0, carry)
            if t + 1 < _TPW:
                cpa = pltpu.async_copy(src(t + 1, 0)[2], bufa, sema)
            cpb.wait()
            carry = _rows_scan(bufb, _H1, _H0, carry)
            if t + 1 < _TPW:
                cpb = pltpu.async_copy(src(t + 1, 1)[2], bufb, semb)
            for lg in range(_NLG):
                idx_v[pl.ds(lg * 16, 16)] = carry[_NLG + lg]
            kt, cgt, _ = src(t, 0)
            pltpu.sync_copy(idx_v, idx_hbm.at[kt, pl.ds(cgt * _CG, _CG)])

    return sc_argmax


def _mlp_t(fe_t, w1_ref, b1_ref, w2t_ref, b2_ref, o_ref):
    h = lax.dot_general(w1_ref[...], fe_t, (((0,), (0,)), ((), ())),
                        preferred_element_type=jnp.float32)
    h = jnp.maximum(h + b1_ref[...], 0.0)
    o = jnp.dot(w2t_ref[...], h, preferred_element_type=jnp.float32)
    o_ref[...] = o + b2_ref[...]


def _fused_body(xt_ref, embt_ref, w1_ref, b1_ref, w2t_ref, b2_ref, o_ref):
    xb = xt_ref[...]
    embt = embt_ref[...]
    fe_parts = []
    for k in range(_K):
        xk = xb[k * _VOCAB:(k + 1) * _VOCAB, :]
        m = jnp.max(xk, axis=0, keepdims=True)
        ii = lax.broadcasted_iota(jnp.int32, xk.shape, 0)
        cand = jnp.where(xk == m, ii, _VOCAB)
        idxk = jnp.min(cand, axis=0, keepdims=True)
        onehot = (ii == idxk).astype(jnp.float32)
        fe_parts.append(
            jnp.dot(embt, onehot, preferred_element_type=jnp.float32))
    fe_t = jnp.concatenate(fe_parts, axis=0)
    _mlp_t(fe_t, w1_ref, b1_ref, w2t_ref, b2_ref, o_ref)


def _fused(xt, embt, w1, b1c, w2t, b2c, ncols):
    grid = ncols // _BB
    return pl.pallas_call(
        _fused_body,
        grid=(grid,),
        in_specs=[
            pl.BlockSpec((_ROW_W, _BB), lambda i: (0, i)),
            pl.BlockSpec((_EMBED, _VOCAB), lambda i: (0, 0)),
            pl.BlockSpec((_K * _EMBED, _HID), lambda i: (0, 0)),
            pl.BlockSpec((_HID, 1), lambda i: (0, 0)),
            pl.BlockSpec((_OUT, _HID), lambda i: (0, 0)),
            pl.BlockSpec((_OUT, 1), lambda i: (0, 0)),
        ],
        out_specs=pl.BlockSpec((_OUT, _BB), lambda i: (0, i)),
        out_shape=jax.ShapeDtypeStruct((_OUT, _B), jnp.float32),
    )(xt, embt, w1, b1c, w2t, b2c)


def _tail_body(prev_ref, idx_ref, embt_ref, w1_ref, b1_ref, w2t_ref, b2_ref,
               o_ref):
    del prev_ref
    idxb = idx_ref[...]
    embt = embt_ref[...]
    fe_parts = []
    for k in range(_K):
        idxk = idxb[k:k + 1, :]
        ii = lax.broadcasted_iota(jnp.int32, (_VOCAB, idxb.shape[1]), 0)
        onehot = (ii == idxk).astype(jnp.float32)
        fe_parts.append(
            jnp.dot(embt, onehot, preferred_element_type=jnp.float32))
    fe_t = jnp.concatenate(fe_parts, axis=0)
    _mlp_t(fe_t, w1_ref, b1_ref, w2t_ref, b2_ref, o_ref)


def _tail_mlp(prev_out_t, idx_t, embt, w1, b1c, w2t, b2c):
    grid = _DS // _BB
    off = _C0 // _BB
    return pl.pallas_call(
        _tail_body,
        grid=(grid,),
        in_specs=[
            pl.BlockSpec(memory_space=pl.ANY),
            pl.BlockSpec((_K, _BB), lambda i: (0, i)),
            pl.BlockSpec((_EMBED, _VOCAB), lambda i: (0, 0)),
            pl.BlockSpec((_K * _EMBED, _HID), lambda i: (0, 0)),
            pl.BlockSpec((_HID, 1), lambda i: (0, 0)),
            pl.BlockSpec((_OUT, _HID), lambda i: (0, 0)),
            pl.BlockSpec((_OUT, 1), lambda i: (0, 0)),
        ],
        out_specs=pl.BlockSpec((_OUT, _BB), lambda i: (0, off + i)),
        out_shape=jax.ShapeDtypeStruct((_OUT, _B), jnp.float32),
        input_output_aliases={0: 0},
    )(prev_out_t, idx_t, embt, w1, b1c, w2t, b2c)


def kernel(x, emb, W1, b1, W2, b2):
    xt = x.T
    embt = emb.T
    w2t = W2.T
    b1c = b1.reshape(_HID, 1)
    b2c = b2.reshape(_OUT, 1)
    idx_t = _make_sc_argmax()(xt)
    head_t = _fused(xt, embt, W1, b1c, w2t, b2c, _C0)
    out_t = _tail_mlp(head_t, idx_t, embt, W1, b1c, w2t, b2c)
    return out_t.T
